# baseline (speedup 1.0000x reference)
import jax
import jax.numpy as jnp
from jax.experimental import pallas as pl
from jax.experimental.pallas import tpu as pltpu

ROW_BLOCK = 400


def _dot_bt(x, w):
    return jax.lax.dot_general(
        x, w, dimension_numbers=(((1,), (1,)), ((), ())),
        preferred_element_type=jnp.float32)


HALF = ROW_BLOCK // 2


def _kgagg_body(a0_ref, a1_ref, e_ref, ws_ref, wn_ref, bs_ref, bn_ref, out_ref):
    i = pl.program_id(0)
    e = e_ref[...]
    b = bs_ref[...] + bn_ref[...]
    wn = wn_ref[...]
    ws = ws_ref[...]
    for h, a_ref in enumerate((a0_ref, a1_ref)):
        neigh = jnp.dot(a_ref[...], e, preferred_element_type=jnp.float32)
        neigh = _dot_bt(neigh, wn)
        e_blk = e_ref[pl.ds(i * ROW_BLOCK + h * HALF, HALF), :]
        x = _dot_bt(e_blk, ws) + neigh + b
        out_ref[pl.ds(h * HALF, HALF), :] = jnp.where(x >= 0.0, x, 0.01 * x)


def kernel(entity_embs, adj_matrix, W_self, b_self, W_neigh, b_neigh):
    n, d_in = entity_embs.shape
    d_out = W_self.shape[0]
    return pl.pallas_call(
        _kgagg_body,
        grid=(n // ROW_BLOCK,),
        in_specs=[
            pl.BlockSpec((ROW_BLOCK // 2, n), lambda i: (2 * i, 0)),
            pl.BlockSpec((ROW_BLOCK // 2, n), lambda i: (2 * i + 1, 0)),
            pl.BlockSpec((n, d_in), lambda i: (0, 0)),
            pl.BlockSpec((d_out, d_in), lambda i: (0, 0)),
            pl.BlockSpec((d_out, d_in), lambda i: (0, 0)),
            pl.BlockSpec((1, d_out), lambda i: (0, 0)),
            pl.BlockSpec((1, d_out), lambda i: (0, 0)),
        ],
        out_specs=pl.BlockSpec((ROW_BLOCK, d_out), lambda i: (i, 0)),
        out_shape=jax.ShapeDtypeStruct((n, d_out), jnp.float32),
        compiler_params=pltpu.CompilerParams(
            dimension_semantics=("parallel",),
        ),
    )(adj_matrix, adj_matrix, entity_embs, W_self, W_neigh,
      b_self.reshape(1, d_out), b_neigh.reshape(1, d_out))


# speedup vs baseline: 1.1001x; 1.1001x over previous
import jax
import jax.numpy as jnp
from jax.experimental import pallas as pl
from jax.experimental.pallas import tpu as pltpu

ROW_BLOCK = 400


def _dot_bt(x, w):
    return jax.lax.dot_general(
        x, w, dimension_numbers=(((1,), (1,)), ((), ())),
        preferred_element_type=jnp.float32)


def _kgagg_body(a_ref, e_ref, ws_ref, wn_ref, bs_ref, bn_ref, out_ref):
    i = pl.program_id(0)
    neigh = jnp.dot(a_ref[...], e_ref[...], preferred_element_type=jnp.float32)
    neigh = _dot_bt(neigh, wn_ref[...])
    e_blk = e_ref[pl.ds(i * ROW_BLOCK, ROW_BLOCK), :]
    self_t = _dot_bt(e_blk, ws_ref[...])
    x = self_t + neigh + (bs_ref[...] + bn_ref[...])
    out_ref[...] = jnp.where(x >= 0.0, x, 0.01 * x)


def kernel(entity_embs, adj_matrix, W_self, b_self, W_neigh, b_neigh):
    n, d_in = entity_embs.shape
    d_out = W_self.shape[0]
    return pl.pallas_call(
        _kgagg_body,
        grid=(n // ROW_BLOCK,),
        in_specs=[
            pl.BlockSpec((ROW_BLOCK, n), lambda i: (i, 0)),
            pl.BlockSpec((n, d_in), lambda i: (0, 0)),
            pl.BlockSpec((d_out, d_in), lambda i: (0, 0)),
            pl.BlockSpec((d_out, d_in), lambda i: (0, 0)),
            pl.BlockSpec((1, d_out), lambda i: (0, 0)),
            pl.BlockSpec((1, d_out), lambda i: (0, 0)),
        ],
        out_specs=pl.BlockSpec((ROW_BLOCK, d_out), lambda i: (i, 0)),
        out_shape=jax.ShapeDtypeStruct((n, d_out), jnp.float32),
        compiler_params=pltpu.CompilerParams(
            dimension_semantics=("parallel",),
        ),
    )(adj_matrix, entity_embs, W_self, W_neigh,
      b_self.reshape(1, d_out), b_neigh.reshape(1, d_out))
